# Initial kernel scaffold; baseline (speedup 1.0000x reference)
import jax
import jax.numpy as jnp
from jax.experimental import pallas as pl


def kernel(x, edge_index, batch, edge_attr, W1, b1, W2, b2, Wc1, bc1, Wc2, bc2, Wn1, bn1, Wn2, bn2, scale):
    raise NotImplementedError("write your pallas kernel here")


# speedup vs baseline: 1.7757x; 1.7757x over previous
import functools

import jax
import jax.numpy as jnp
from jax import lax
from jax.experimental import pallas as pl
from jax.experimental.pallas import tpu as pltpu
from jax.experimental.pallas import tpu_sc as plsc

N = 10000
E = 160000
POS = 3
FEATS = 128
EA = 16
MSG = 16
EIN = FEATS * 2 + EA + 1
HID = 2 * EIN
CW = 16
PW = 32

NC = 2
NS = 16
NW = NC * NS

E_PAD = 163840
ROWS = E_PAD // 128
ROWS_W = ROWS // NW
KR = 4
CHUNKS = ROWS_W // KR

N_PAD = 10240
NROWS_S = N_PAD // NS

BE = 640
BN = 1000


def _silu(t):
    return t * jax.nn.sigmoid(t)


def _gather_body(feats_hbm, coors_hbm, srcx_hbm, dstx_hbm,
                 xgs_hbm, xgd_hbm, cgs_hbm, cgd_hbm,
                 sidx, didx, fbuf, cbuf, sem):
    wid = lax.axis_index("s") * NC + lax.axis_index("c")
    row0 = wid * ROWS_W

    def chunk(it, _):
        r0 = row0 + it * KR
        pltpu.sync_copy(srcx_hbm.at[pl.ds(r0, KR)], sidx)
        pltpu.sync_copy(dstx_hbm.at[pl.ds(r0, KR)], didx)
        e0 = r0 * 128
        for j in range(KR):
            pltpu.async_copy(feats_hbm.at[sidx.at[j]],
                             fbuf.at[pl.ds(j * 128, 128)], sem).wait()
        pltpu.sync_copy(fbuf, xgs_hbm.at[pl.ds(e0, KR * 128)])
        for j in range(KR):
            pltpu.async_copy(feats_hbm.at[didx.at[j]],
                             fbuf.at[pl.ds(j * 128, 128)], sem).wait()
        pltpu.sync_copy(fbuf, xgd_hbm.at[pl.ds(e0, KR * 128)])
        for j in range(KR):
            pltpu.async_copy(coors_hbm.at[sidx.at[j]],
                             cbuf.at[pl.ds(j * 128, 128)], sem).wait()
        pltpu.sync_copy(cbuf, cgs_hbm.at[pl.ds(e0, KR * 128)])
        for j in range(KR):
            pltpu.async_copy(coors_hbm.at[didx.at[j]],
                             cbuf.at[pl.ds(j * 128, 128)], sem).wait()
        pltpu.sync_copy(cbuf, cgd_hbm.at[pl.ds(e0, KR * 128)])
        return 0

    lax.fori_loop(0, CHUNKS, chunk, 0)


@functools.cache
def _gather_kernel():
    return pl.kernel(
        _gather_body,
        out_type=(
            jax.ShapeDtypeStruct((E_PAD, FEATS), jnp.float32),
            jax.ShapeDtypeStruct((E_PAD, FEATS), jnp.float32),
            jax.ShapeDtypeStruct((E_PAD, CW), jnp.float32),
            jax.ShapeDtypeStruct((E_PAD, CW), jnp.float32),
        ),
        mesh=plsc.VectorSubcoreMesh(
            core_axis_name="c", subcore_axis_name="s",
            num_cores=NC, num_subcores=NS),
        scratch_types=[
            pltpu.VMEM((KR, 128), jnp.int32),
            pltpu.VMEM((KR, 128), jnp.int32),
            pltpu.VMEM((KR * 128, FEATS), jnp.float32),
            pltpu.VMEM((KR * 128, CW), jnp.float32),
            pltpu.SemaphoreType.DMA,
        ],
        compiler_params=pltpu.CompilerParams(use_tc_tiling_on_sc=False),
    )


def _gather_call(*args):
    return _gather_kernel()(*args)


def _edge_body(xgs, xgd, cgs, cgd, ea,
               w1d, w1s, w1ea, w1rd, b1, w2, b2, wc1, bc1, wc2, bc2,
               out):
    i = pl.program_id(0)
    rel = cgs[...] - cgd[...]
    rd = jnp.sqrt(jnp.sum(rel * rel, axis=1, keepdims=True))
    h = (jnp.dot(xgd[...], w1d[...], preferred_element_type=jnp.float32)
         + jnp.dot(xgs[...], w1s[...], preferred_element_type=jnp.float32)
         + jnp.dot(ea[...], w1ea[...], preferred_element_type=jnp.float32)
         + rd * w1rd[...] + b1[...])
    h = _silu(h)
    m = _silu(jnp.dot(h, w2[...], preferred_element_type=jnp.float32) + b2[...])
    cw = (jnp.dot(_silu(jnp.dot(m, wc1[...], preferred_element_type=jnp.float32)
                        + bc1[...]),
                  wc2[...], preferred_element_type=jnp.float32) + bc2[...])
    packed = jnp.concatenate(
        [m, cw, rel[:, :3], jnp.zeros((BE, PW - MSG - 4), jnp.float32)], axis=1)
    eids = i * BE + lax.broadcasted_iota(jnp.int32, (BE, 1), 0)
    out[...] = jnp.where(eids < E, packed, 0.0)


def _edge_call(xgs, xgd, cgs, cgd, ea, w1d, w1s, w1ea, w1rd, b1, w2, b2,
               wc1, bc1, wc2, bc2):
    full = lambda shape: pl.BlockSpec(shape, lambda i: (0, 0))
    return pl.pallas_call(
        _edge_body,
        grid=(E_PAD // BE,),
        in_specs=[
            pl.BlockSpec((BE, FEATS), lambda i: (i, 0)),
            pl.BlockSpec((BE, FEATS), lambda i: (i, 0)),
            pl.BlockSpec((BE, CW), lambda i: (i, 0)),
            pl.BlockSpec((BE, CW), lambda i: (i, 0)),
            pl.BlockSpec((BE, EA), lambda i: (i, 0)),
            full((FEATS, HID)),
            full((FEATS, HID)),
            full((EA, HID)),
            full((1, HID)),
            full((1, HID)),
            full((HID, MSG)),
            full((1, MSG)),
            full((MSG, 4 * MSG)),
            full((1, 4 * MSG)),
            full((4 * MSG, 1)),
            full((1, 1)),
        ],
        out_specs=pl.BlockSpec((BE, PW), lambda i: (i, 0)),
        out_shape=jax.ShapeDtypeStruct((E_PAD, PW), jnp.float32),
    )(xgs, xgd, cgs, cgd, ea, w1d, w1s, w1ea, w1rd, b1, w2, b2,
      wc1, bc1, wc2, bc2)


def _scatter_body(mcw_hbm, dstx_hbm, zeros_hbm, p0_hbm, p1_hbm,
                  didx, mbuf, acc):
    cid = lax.axis_index("c")
    sid = lax.axis_index("s")
    pltpu.sync_copy(zeros_hbm.at[pl.ds(sid * NROWS_S, NROWS_S)],
                    acc.at[pl.ds(sid * NROWS_S, NROWS_S)])
    plsc.subcore_barrier()
    row0 = cid * (ROWS // NC) + sid * ROWS_W

    def chunk(it, _):
        r0 = row0 + it * KR
        pltpu.sync_copy(dstx_hbm.at[pl.ds(r0, KR)], didx)
        pltpu.sync_copy(mcw_hbm.at[pl.ds(r0 * 128, KR * 128)], mbuf)
        for j in range(KR):
            pltpu.sync_copy(mbuf.at[pl.ds(j * 128, 128)],
                            acc.at[didx.at[j]], add=True)
        return 0

    lax.fori_loop(0, CHUNKS, chunk, 0)
    plsc.subcore_barrier()

    @pl.when(cid == 0)
    def _():
        pltpu.sync_copy(acc.at[pl.ds(sid * NROWS_S, NROWS_S)],
                        p0_hbm.at[pl.ds(sid * NROWS_S, NROWS_S)])

    @pl.when(cid == 1)
    def _():
        pltpu.sync_copy(acc.at[pl.ds(sid * NROWS_S, NROWS_S)],
                        p1_hbm.at[pl.ds(sid * NROWS_S, NROWS_S)])


@functools.cache
def _scatter_kernel():
    return pl.kernel(
        _scatter_body,
        out_type=(
            jax.ShapeDtypeStruct((N_PAD, PW), jnp.float32),
            jax.ShapeDtypeStruct((N_PAD, PW), jnp.float32),
        ),
        mesh=plsc.VectorSubcoreMesh(
            core_axis_name="c", subcore_axis_name="s",
            num_cores=NC, num_subcores=NS),
        scratch_types=[
            pltpu.VMEM((KR, 128), jnp.int32),
            pltpu.VMEM((KR * 128, PW), jnp.float32),
            pltpu.VMEM_SHARED((N_PAD, PW), jnp.float32),
        ],
        compiler_params=pltpu.CompilerParams(use_tc_tiling_on_sc=False),
    )


def _scatter_call(*args):
    return _scatter_kernel()(*args)


def _node_body(feats, coors, p0, p1, wn1f, wn1m, bn1, wn2, bn2, scl,
               coors_out, feats_out):
    m_i = p0[:, :MSG] + p1[:, :MSG]
    cwsum = p0[:, MSG:MSG + 1] + p1[:, MSG:MSG + 1]
    cri = p0[:, MSG + 1:MSG + 4] + p1[:, MSG + 1:MSG + 4]
    cw = jnp.tanh(cwsum)
    nrm = jnp.sqrt(jnp.sum(cri * cri, axis=1, keepdims=True))
    crin = cri / jnp.maximum(nrm, 1e-12) * scl[0, 0]
    delta = cw * crin
    delta_w = jnp.concatenate(
        [delta, jnp.zeros((BN, CW - 3), jnp.float32)], axis=1)
    coors_out[...] = coors[...] + delta_w
    h = _silu(jnp.dot(feats[...], wn1f[...], preferred_element_type=jnp.float32)
              + jnp.dot(m_i, wn1m[...], preferred_element_type=jnp.float32)
              + bn1[...])
    feats_out[...] = (jnp.dot(h, wn2[...], preferred_element_type=jnp.float32)
                      + bn2[...] + feats[...])


def _node_call(feats, coors, p0, p1, wn1f, wn1m, bn1, wn2, bn2, scl):
    full = lambda shape: pl.BlockSpec(shape, lambda i: (0, 0))
    return pl.pallas_call(
        _node_body,
        grid=(N // BN,),
        in_specs=[
            pl.BlockSpec((BN, FEATS), lambda i: (i, 0)),
            pl.BlockSpec((BN, CW), lambda i: (i, 0)),
            pl.BlockSpec((BN, PW), lambda i: (i, 0)),
            pl.BlockSpec((BN, PW), lambda i: (i, 0)),
            full((FEATS, 2 * FEATS)),
            full((MSG, 2 * FEATS)),
            full((1, 2 * FEATS)),
            full((2 * FEATS, FEATS)),
            full((1, FEATS)),
            full((1, 1)),
        ],
        out_specs=[
            pl.BlockSpec((BN, CW), lambda i: (i, 0)),
            pl.BlockSpec((BN, FEATS), lambda i: (i, 0)),
        ],
        out_shape=[
            jax.ShapeDtypeStruct((N, CW), jnp.float32),
            jax.ShapeDtypeStruct((N, FEATS), jnp.float32),
        ],
    )(feats, coors, p0, p1, wn1f, wn1m, bn1, wn2, bn2, scl)


def kernel(x, edge_index, batch, edge_attr, W1, b1, W2, b2, Wc1, bc1, Wc2,
           bc2, Wn1, bn1, Wn2, bn2, scale):
    src = edge_index[0]
    dst = edge_index[1]
    pad = E_PAD - E
    srcx = jnp.concatenate([src, jnp.zeros((pad,), jnp.int32)]).reshape(ROWS, 128)
    dstx = jnp.concatenate([dst, jnp.zeros((pad,), jnp.int32)]).reshape(ROWS, 128)
    eap = jnp.concatenate([edge_attr, jnp.zeros((pad, EA), jnp.float32)], axis=0)
    zeros_n = jnp.zeros((N_PAD, PW), jnp.float32)

    coors = jnp.pad(x[:, :POS], ((0, 0), (0, CW - POS)))
    feats = x[:, POS:]

    L = W1.shape[0]
    for l in range(L):
        w1 = W1[l]
        xgs, xgd, cgs, cgd = _gather_call(feats, coors, srcx, dstx)
        mcw = _edge_call(
            xgs, xgd, cgs, cgd, eap,
            w1[:FEATS], w1[FEATS:2 * FEATS], w1[2 * FEATS:2 * FEATS + EA],
            w1[2 * FEATS + EA:], b1[l][None, :], W2[l], b2[l][None, :],
            Wc1[l], bc1[l][None, :], Wc2[l], bc2[l][None, :])
        p0, p1 = _scatter_call(mcw, dstx, zeros_n)
        coors, feats = _node_call(
            feats, coors, p0, p1,
            Wn1[l][:FEATS], Wn1[l][FEATS:], bn1[l][None, :], Wn2[l],
            bn2[l][None, :], scale[l][None, :])

    return jnp.concatenate([coors[:, :POS], feats], axis=1)


# speedup vs baseline: 1.8666x; 1.0512x over previous
import functools

import jax
import jax.numpy as jnp
from jax import lax
from jax.experimental import pallas as pl
from jax.experimental.pallas import tpu as pltpu
from jax.experimental.pallas import tpu_sc as plsc

N = 10000
E = 160000
POS = 3
FEATS = 128
EA = 16
MSG = 16
EIN = FEATS * 2 + EA + 1
HID = 2 * EIN
CW = 16
PW = 32

NC = 2
NS = 16
NW = NC * NS

E_PAD = 163840
ROWS = E_PAD // 128
ROWS_W = ROWS // NW
KR = 4
CHUNKS = ROWS_W // KR

N_PAD = 10240
NROWS_S = N_PAD // NS

BE = 640
BN = 1000


def _silu(t):
    return t * jax.nn.sigmoid(t)


def _gather_body(feats_hbm, coors_hbm, srcx_hbm, dstx_hbm,
                 xgs_hbm, xgd_hbm, cgs_hbm, cgd_hbm,
                 sidx, didx, fbs, fbd, cbs, cbd,
                 gs1, gs2, gs3, gs4, ws1, ws2, ws3, ws4):
    wid = lax.axis_index("s") * NC + lax.axis_index("c")
    row0 = wid * ROWS_W
    pltpu.sync_copy(srcx_hbm.at[pl.ds(row0, ROWS_W)], sidx)
    pltpu.sync_copy(dstx_hbm.at[pl.ds(row0, ROWS_W)], didx)

    def fire_g(itv, b):
        pltpu.async_copy(feats_hbm.at[sidx.at[itv]], fbs.at[b], gs1)
        pltpu.async_copy(feats_hbm.at[didx.at[itv]], fbd.at[b], gs2)
        pltpu.async_copy(coors_hbm.at[sidx.at[itv]], cbs.at[b], gs3)
        pltpu.async_copy(coors_hbm.at[didx.at[itv]], cbd.at[b], gs4)

    def drain_g():
        pltpu.make_async_copy(feats_hbm.at[sidx.at[0]], fbs.at[0], gs1).wait()
        pltpu.make_async_copy(feats_hbm.at[didx.at[0]], fbd.at[0], gs2).wait()
        pltpu.make_async_copy(coors_hbm.at[sidx.at[0]], cbs.at[0], gs3).wait()
        pltpu.make_async_copy(coors_hbm.at[didx.at[0]], cbd.at[0], gs4).wait()

    def fire_w(itv, b):
        e0 = (row0 + itv) * 128
        pltpu.async_copy(fbs.at[b], xgs_hbm.at[pl.ds(e0, 128)], ws1)
        pltpu.async_copy(fbd.at[b], xgd_hbm.at[pl.ds(e0, 128)], ws2)
        pltpu.async_copy(cbs.at[b], cgs_hbm.at[pl.ds(e0, 128)], ws3)
        pltpu.async_copy(cbd.at[b], cgd_hbm.at[pl.ds(e0, 128)], ws4)

    def drain_w():
        pltpu.make_async_copy(fbs.at[0], xgs_hbm.at[pl.ds(0, 128)], ws1).wait()
        pltpu.make_async_copy(fbd.at[0], xgd_hbm.at[pl.ds(0, 128)], ws2).wait()
        pltpu.make_async_copy(cbs.at[0], cgs_hbm.at[pl.ds(0, 128)], ws3).wait()
        pltpu.make_async_copy(cbd.at[0], cgd_hbm.at[pl.ds(0, 128)], ws4).wait()

    fire_g(0, 0)

    def body(it, _):
        @pl.when(it <= ROWS_W - 2)
        def _():
            @pl.when(it >= 1)
            def _():
                drain_w()
            fire_g(it + 1, (it + 1) % 2)
        drain_g()
        fire_w(it, it % 2)
        return 0

    lax.fori_loop(0, ROWS_W, body, 0)
    drain_w()
    drain_w()


@functools.cache
def _gather_kernel():
    return pl.kernel(
        _gather_body,
        out_type=(
            jax.ShapeDtypeStruct((E_PAD, FEATS), jnp.bfloat16),
            jax.ShapeDtypeStruct((E_PAD, FEATS), jnp.bfloat16),
            jax.ShapeDtypeStruct((E_PAD, CW), jnp.float32),
            jax.ShapeDtypeStruct((E_PAD, CW), jnp.float32),
        ),
        mesh=plsc.VectorSubcoreMesh(
            core_axis_name="c", subcore_axis_name="s",
            num_cores=NC, num_subcores=NS),
        scratch_types=[
            pltpu.VMEM((ROWS_W, 128), jnp.int32),
            pltpu.VMEM((ROWS_W, 128), jnp.int32),
            pltpu.VMEM((2, 128, FEATS), jnp.bfloat16),
            pltpu.VMEM((2, 128, FEATS), jnp.bfloat16),
            pltpu.VMEM((2, 128, CW), jnp.float32),
            pltpu.VMEM((2, 128, CW), jnp.float32),
            pltpu.SemaphoreType.DMA,
            pltpu.SemaphoreType.DMA,
            pltpu.SemaphoreType.DMA,
            pltpu.SemaphoreType.DMA,
            pltpu.SemaphoreType.DMA,
            pltpu.SemaphoreType.DMA,
            pltpu.SemaphoreType.DMA,
            pltpu.SemaphoreType.DMA,
        ],
        compiler_params=pltpu.CompilerParams(use_tc_tiling_on_sc=False),
    )


def _gather_call(*args):
    return _gather_kernel()(*args)


def _edge_body(xgs, xgd, cgs, cgd, ea,
               w1d, w1s, w1ea, w1rd, b1, w2, b2, wc1, bc1, wc2, bc2,
               out):
    i = pl.program_id(0)
    rel = cgs[...] - cgd[...]
    rd = jnp.sqrt(jnp.sum(rel * rel, axis=1, keepdims=True))
    h = (jnp.dot(xgd[...], w1d[...], preferred_element_type=jnp.float32)
         + jnp.dot(xgs[...], w1s[...], preferred_element_type=jnp.float32)
         + jnp.dot(ea[...], w1ea[...], preferred_element_type=jnp.float32)
         + rd * w1rd[...] + b1[...])
    h = _silu(h)
    m = _silu(jnp.dot(h, w2[...], preferred_element_type=jnp.float32) + b2[...])
    cw = (jnp.dot(_silu(jnp.dot(m, wc1[...], preferred_element_type=jnp.float32)
                        + bc1[...]),
                  wc2[...], preferred_element_type=jnp.float32) + bc2[...])
    packed = jnp.concatenate(
        [m, cw, rel[:, :3], jnp.zeros((BE, PW - MSG - 4), jnp.float32)], axis=1)
    eids = i * BE + lax.broadcasted_iota(jnp.int32, (BE, 1), 0)
    out[...] = jnp.where(eids < E, packed, 0.0)


def _edge_call(xgs, xgd, cgs, cgd, ea, w1d, w1s, w1ea, w1rd, b1, w2, b2,
               wc1, bc1, wc2, bc2):
    full = lambda shape: pl.BlockSpec(shape, lambda i: (0, 0))
    return pl.pallas_call(
        _edge_body,
        grid=(E_PAD // BE,),
        in_specs=[
            pl.BlockSpec((BE, FEATS), lambda i: (i, 0)),
            pl.BlockSpec((BE, FEATS), lambda i: (i, 0)),
            pl.BlockSpec((BE, CW), lambda i: (i, 0)),
            pl.BlockSpec((BE, CW), lambda i: (i, 0)),
            pl.BlockSpec((BE, EA), lambda i: (i, 0)),
            full((FEATS, HID)),
            full((FEATS, HID)),
            full((EA, HID)),
            full((1, HID)),
            full((1, HID)),
            full((HID, MSG)),
            full((1, MSG)),
            full((MSG, 4 * MSG)),
            full((1, 4 * MSG)),
            full((4 * MSG, 1)),
            full((1, 1)),
        ],
        out_specs=pl.BlockSpec((BE, PW), lambda i: (i, 0)),
        out_shape=jax.ShapeDtypeStruct((E_PAD, PW), jnp.float32),
    )(xgs, xgd, cgs, cgd, ea, w1d, w1s, w1ea, w1rd, b1, w2, b2,
      wc1, bc1, wc2, bc2)


def _scatter_body(mcw_hbm, dstx_hbm, zeros_hbm, p0_hbm, p1_hbm,
                  didx, mbuf, acc):
    cid = lax.axis_index("c")
    sid = lax.axis_index("s")
    pltpu.sync_copy(zeros_hbm.at[pl.ds(sid * NROWS_S, NROWS_S)],
                    acc.at[pl.ds(sid * NROWS_S, NROWS_S)])
    plsc.subcore_barrier()
    row0 = cid * (ROWS // NC) + sid * ROWS_W

    def chunk(it, _):
        r0 = row0 + it * KR
        pltpu.sync_copy(dstx_hbm.at[pl.ds(r0, KR)], didx)
        pltpu.sync_copy(mcw_hbm.at[pl.ds(r0 * 128, KR * 128)], mbuf)
        for j in range(KR):
            pltpu.sync_copy(mbuf.at[pl.ds(j * 128, 128)],
                            acc.at[didx.at[j]], add=True)
        return 0

    lax.fori_loop(0, CHUNKS, chunk, 0)
    plsc.subcore_barrier()

    @pl.when(cid == 0)
    def _():
        pltpu.sync_copy(acc.at[pl.ds(sid * NROWS_S, NROWS_S)],
                        p0_hbm.at[pl.ds(sid * NROWS_S, NROWS_S)])

    @pl.when(cid == 1)
    def _():
        pltpu.sync_copy(acc.at[pl.ds(sid * NROWS_S, NROWS_S)],
                        p1_hbm.at[pl.ds(sid * NROWS_S, NROWS_S)])


@functools.cache
def _scatter_kernel():
    return pl.kernel(
        _scatter_body,
        out_type=(
            jax.ShapeDtypeStruct((N_PAD, PW), jnp.float32),
            jax.ShapeDtypeStruct((N_PAD, PW), jnp.float32),
        ),
        mesh=plsc.VectorSubcoreMesh(
            core_axis_name="c", subcore_axis_name="s",
            num_cores=NC, num_subcores=NS),
        scratch_types=[
            pltpu.VMEM((KR, 128), jnp.int32),
            pltpu.VMEM((KR * 128, PW), jnp.float32),
            pltpu.VMEM_SHARED((N_PAD, PW), jnp.float32),
        ],
        compiler_params=pltpu.CompilerParams(use_tc_tiling_on_sc=False),
    )


def _scatter_call(*args):
    return _scatter_kernel()(*args)


def _node_body(feats, coors, p0, p1, wn1f, wn1m, bn1, wn2, bn2, scl,
               coors_out, feats_out, featsb_out):
    m_i = p0[:, :MSG] + p1[:, :MSG]
    cwsum = p0[:, MSG:MSG + 1] + p1[:, MSG:MSG + 1]
    cri = p0[:, MSG + 1:MSG + 4] + p1[:, MSG + 1:MSG + 4]
    cw = jnp.tanh(cwsum)
    nrm = jnp.sqrt(jnp.sum(cri * cri, axis=1, keepdims=True))
    crin = cri / jnp.maximum(nrm, 1e-12) * scl[0, 0]
    delta = cw * crin
    delta_w = jnp.concatenate(
        [delta, jnp.zeros((BN, CW - 3), jnp.float32)], axis=1)
    coors_out[...] = coors[...] + delta_w
    h = _silu(jnp.dot(feats[...], wn1f[...], preferred_element_type=jnp.float32)
              + jnp.dot(m_i, wn1m[...], preferred_element_type=jnp.float32)
              + bn1[...])
    f_new = (jnp.dot(h, wn2[...], preferred_element_type=jnp.float32)
             + bn2[...] + feats[...])
    feats_out[...] = f_new
    featsb_out[...] = f_new.astype(jnp.bfloat16)


def _node_call(feats, coors, p0, p1, wn1f, wn1m, bn1, wn2, bn2, scl):
    full = lambda shape: pl.BlockSpec(shape, lambda i: (0, 0))
    return pl.pallas_call(
        _node_body,
        grid=(N // BN,),
        in_specs=[
            pl.BlockSpec((BN, FEATS), lambda i: (i, 0)),
            pl.BlockSpec((BN, CW), lambda i: (i, 0)),
            pl.BlockSpec((BN, PW), lambda i: (i, 0)),
            pl.BlockSpec((BN, PW), lambda i: (i, 0)),
            full((FEATS, 2 * FEATS)),
            full((MSG, 2 * FEATS)),
            full((1, 2 * FEATS)),
            full((2 * FEATS, FEATS)),
            full((1, FEATS)),
            full((1, 1)),
        ],
        out_specs=[
            pl.BlockSpec((BN, CW), lambda i: (i, 0)),
            pl.BlockSpec((BN, FEATS), lambda i: (i, 0)),
            pl.BlockSpec((BN, FEATS), lambda i: (i, 0)),
        ],
        out_shape=[
            jax.ShapeDtypeStruct((N, CW), jnp.float32),
            jax.ShapeDtypeStruct((N, FEATS), jnp.float32),
            jax.ShapeDtypeStruct((N, FEATS), jnp.bfloat16),
        ],
    )(feats, coors, p0, p1, wn1f, wn1m, bn1, wn2, bn2, scl)


def kernel(x, edge_index, batch, edge_attr, W1, b1, W2, b2, Wc1, bc1, Wc2,
           bc2, Wn1, bn1, Wn2, bn2, scale):
    src = edge_index[0]
    dst = edge_index[1]
    pad = E_PAD - E
    srcx = jnp.concatenate([src, jnp.zeros((pad,), jnp.int32)]).reshape(ROWS, 128)
    dstx = jnp.concatenate([dst, jnp.zeros((pad,), jnp.int32)]).reshape(ROWS, 128)
    eap = jnp.concatenate([edge_attr, jnp.zeros((pad, EA), jnp.float32)], axis=0)
    zeros_n = jnp.zeros((N_PAD, PW), jnp.float32)

    coors = jnp.pad(x[:, :POS], ((0, 0), (0, CW - POS)))
    feats = x[:, POS:]
    feats_bf = feats.astype(jnp.bfloat16)

    L = W1.shape[0]
    for l in range(L):
        w1 = W1[l]
        xgs, xgd, cgs, cgd = _gather_call(feats_bf, coors, srcx, dstx)
        mcw = _edge_call(
            xgs, xgd, cgs, cgd, eap,
            w1[:FEATS].astype(jnp.bfloat16),
            w1[FEATS:2 * FEATS].astype(jnp.bfloat16),
            w1[2 * FEATS:2 * FEATS + EA],
            w1[2 * FEATS + EA:], b1[l][None, :], W2[l], b2[l][None, :],
            Wc1[l], bc1[l][None, :], Wc2[l], bc2[l][None, :])
        p0, p1 = _scatter_call(mcw, dstx, zeros_n)
        coors, feats, feats_bf = _node_call(
            feats, coors, p0, p1,
            Wn1[l][:FEATS], Wn1[l][FEATS:], bn1[l][None, :], Wn2[l],
            bn2[l][None, :], scale[l][None, :])

    return jnp.concatenate([coors[:, :POS], feats], axis=1)
